# baseline (speedup 1.0000x reference)
import functools

import jax
import jax.numpy as jnp
from jax import lax
from jax.experimental import pallas as pl
from jax.experimental.pallas import tpu as pltpu
from jax.experimental.pallas import tpu_sc as plsc

B = 64
F = 16384
C = 16
CARD = 4
OUT_F = F // CARD

NUM_CORES = 2
NUM_SUBCORES = 16
NW = NUM_CORES * NUM_SUBCORES
IN_ROWS = B * F
OUT_ROWS = B * OUT_F
IN_PER_W = IN_ROWS // NW
OUT_PER_W = OUT_ROWS // NW

CH_OUT = 512
CH_IN = CH_OUT * CARD
NCHUNK = IN_PER_W // CH_IN


def _make_sc_kernel():
    mesh = plsc.VectorSubcoreMesh(
        core_axis_name="c", subcore_axis_name="s",
        num_cores=NUM_CORES, num_subcores=NUM_SUBCORES)

    @functools.partial(
        pl.kernel,
        out_type=jax.ShapeDtypeStruct((OUT_ROWS, C), jnp.float32),
        mesh=mesh,
        compiler_params=pltpu.CompilerParams(use_tc_tiling_on_sc=False),
        scratch_types=[
            pltpu.VMEM((CH_IN, C), jnp.float32),
            pltpu.VMEM((CH_IN, C), jnp.float32),
            pltpu.VMEM((CH_OUT, C), jnp.float32),
            pltpu.VMEM((CH_OUT, C), jnp.float32),
            pltpu.SemaphoreType.DMA,
            pltpu.SemaphoreType.DMA,
            pltpu.SemaphoreType.DMA,
            pltpu.SemaphoreType.DMA,
        ],
    )
    def sc_kernel(x_hbm, out_hbm, in0, in1, out0, out1,
                  sem_in0, sem_in1, sem_out0, sem_out1):
        wid = lax.axis_index("s") * NUM_CORES + lax.axis_index("c")
        in_base = wid * IN_PER_W
        out_base = wid * OUT_PER_W
        in_bufs = (in0, in1)
        out_bufs = (out0, out1)
        sem_ins = (sem_in0, sem_in1)
        sem_outs = (sem_out0, sem_out1)

        def start_in(c):
            b = c & 1
            return pltpu.async_copy(
                x_hbm.at[pl.ds(in_base + c * CH_IN, CH_IN)],
                in_bufs[b], sem_ins[b])

        def start_out(c):
            b = c & 1
            return pltpu.async_copy(
                out_bufs[b],
                out_hbm.at[pl.ds(out_base + c * CH_OUT, CH_OUT)],
                sem_outs[b])

        def compute(c):
            b = c & 1
            iv = in_bufs[b]
            ov = out_bufs[b]

            def body(i, _):
                base = i * CARD
                s = iv[base, :] + iv[base + 1, :]
                s = s + iv[base + 2, :]
                s = s + iv[base + 3, :]
                ov[i, :] = s
                return 0

            lax.fori_loop(0, CH_OUT, body, 0, unroll=4)

        in_h = [None] * NCHUNK
        out_h = [None] * NCHUNK
        in_h[0] = start_in(0)
        for c in range(NCHUNK):
            if c + 1 < NCHUNK:
                in_h[c + 1] = start_in(c + 1)
            in_h[c].wait()
            if c >= 2:
                out_h[c - 2].wait()
            compute(c)
            out_h[c] = start_out(c)
        out_h[NCHUNK - 2].wait()
        out_h[NCHUNK - 1].wait()

    return sc_kernel


_SC_KERNEL = _make_sc_kernel()


@jax.jit
def kernel(x):
    xf = x.reshape(IN_ROWS, C)
    out = _SC_KERNEL(xf)
    return out.reshape(B, OUT_F, C)


# speedup vs baseline: 1.0005x; 1.0005x over previous
import functools

import jax
import jax.numpy as jnp
from jax import lax
from jax.experimental import pallas as pl
from jax.experimental.pallas import tpu as pltpu
from jax.experimental.pallas import tpu_sc as plsc

B = 64
F = 16384
C = 16
CARD = 4
OUT_F = F // CARD

NUM_CORES = 2
NUM_SUBCORES = 16
NW = NUM_CORES * NUM_SUBCORES
IN_WORDS = B * F * C
OUT_WORDS = B * OUT_F * C
IN_PER_W = IN_WORDS // NW
OUT_PER_W = OUT_WORDS // NW

CH_OUT = 512
CH_IN_W = CH_OUT * CARD * C
CH_OUT_W = CH_OUT * C
NCHUNK = IN_PER_W // CH_IN_W


def _make_sc_kernel():
    mesh = plsc.VectorSubcoreMesh(
        core_axis_name="c", subcore_axis_name="s",
        num_cores=NUM_CORES, num_subcores=NUM_SUBCORES)

    @functools.partial(
        pl.kernel,
        out_type=jax.ShapeDtypeStruct((OUT_WORDS,), jnp.float32),
        mesh=mesh,
        compiler_params=pltpu.CompilerParams(use_tc_tiling_on_sc=True),
        scratch_types=[
            pltpu.VMEM((CH_IN_W,), jnp.float32),
            pltpu.VMEM((CH_IN_W,), jnp.float32),
            pltpu.VMEM((CH_OUT_W,), jnp.float32),
            pltpu.VMEM((CH_OUT_W,), jnp.float32),
            pltpu.SemaphoreType.DMA,
            pltpu.SemaphoreType.DMA,
            pltpu.SemaphoreType.DMA,
            pltpu.SemaphoreType.DMA,
        ],
    )
    def sc_kernel(x_hbm, out_hbm, in0, in1, out0, out1,
                  sem_in0, sem_in1, sem_out0, sem_out1):
        wid = lax.axis_index("s") * NUM_CORES + lax.axis_index("c")
        in_base = wid * IN_PER_W
        out_base = wid * OUT_PER_W
        in_bufs = (in0, in1)
        out_bufs = (out0, out1)
        sem_ins = (sem_in0, sem_in1)
        sem_outs = (sem_out0, sem_out1)

        def start_in(c):
            b = c & 1
            return pltpu.async_copy(
                x_hbm.at[pl.ds(in_base + c * CH_IN_W, CH_IN_W)],
                in_bufs[b], sem_ins[b])

        def start_out(c):
            b = c & 1
            return pltpu.async_copy(
                out_bufs[b],
                out_hbm.at[pl.ds(out_base + c * CH_OUT_W, CH_OUT_W)],
                sem_outs[b])

        def compute(c):
            b = c & 1
            iv = in_bufs[b]
            ov = out_bufs[b]

            def body(i, _):
                ib = i * (CARD * C)
                s = iv[pl.ds(ib, C)] + iv[pl.ds(ib + C, C)]
                s = s + iv[pl.ds(ib + 2 * C, C)]
                s = s + iv[pl.ds(ib + 3 * C, C)]
                ov[pl.ds(i * C, C)] = s
                return 0

            lax.fori_loop(0, CH_OUT, body, 0, unroll=4)

        in_h = [None] * NCHUNK
        out_h = [None] * NCHUNK
        in_h[0] = start_in(0)
        for c in range(NCHUNK):
            if c + 1 < NCHUNK:
                in_h[c + 1] = start_in(c + 1)
            in_h[c].wait()
            if c >= 2:
                out_h[c - 2].wait()
            compute(c)
            out_h[c] = start_out(c)
        out_h[NCHUNK - 2].wait()
        out_h[NCHUNK - 1].wait()

    return sc_kernel


_SC_KERNEL = _make_sc_kernel()


@jax.jit
def kernel(x):
    xf = x.reshape(IN_WORDS)
    out = _SC_KERNEL(xf)
    return out.reshape(B, OUT_F, C)


# speedup vs baseline: 6.2281x; 6.2252x over previous
import functools

import jax
import jax.numpy as jnp
from jax import lax
from jax.experimental import pallas as pl
from jax.experimental.pallas import tpu as pltpu
from jax.experimental.pallas import tpu_sc as plsc

B = 64
F = 16384
C = 16
CARD = 4
OUT_F = F // CARD

NUM_CORES = 2
NUM_SUBCORES = 16
NW = NUM_CORES * NUM_SUBCORES
IN_WORDS = B * F * C
OUT_WORDS = B * OUT_F * C
IN_PER_W = IN_WORDS // NW
OUT_PER_W = OUT_WORDS // NW

CH_IN_W = 32768
CH_OUT_W = CH_IN_W // CARD
NGROUP = CH_OUT_W // 16
NCHUNK = IN_PER_W // CH_IN_W


def _make_sc_kernel():
    mesh = plsc.VectorSubcoreMesh(
        core_axis_name="c", subcore_axis_name="s",
        num_cores=NUM_CORES, num_subcores=NUM_SUBCORES)

    @functools.partial(
        pl.kernel,
        out_type=jax.ShapeDtypeStruct((OUT_WORDS,), jnp.float32),
        mesh=mesh,
        compiler_params=pltpu.CompilerParams(
            use_tc_tiling_on_sc=False, needs_layout_passes=False),
        scratch_types=[
            pltpu.VMEM((CH_IN_W,), jnp.float32),
            pltpu.VMEM((CH_IN_W,), jnp.float32),
            pltpu.VMEM((CH_OUT_W,), jnp.float32),
            pltpu.VMEM((CH_OUT_W,), jnp.float32),
            pltpu.SemaphoreType.DMA,
            pltpu.SemaphoreType.DMA,
            pltpu.SemaphoreType.DMA,
            pltpu.SemaphoreType.DMA,
        ],
    )
    def sc_kernel(x_hbm, out_hbm, in0, in1, out0, out1,
                  sem_in0, sem_in1, sem_out0, sem_out1):
        wid = lax.axis_index("s") * NUM_CORES + lax.axis_index("c")
        in_base = wid * IN_PER_W
        out_base = wid * OUT_PER_W
        in_bufs = (in0, in1)
        out_bufs = (out0, out1)
        sem_ins = (sem_in0, sem_in1)
        sem_outs = (sem_out0, sem_out1)

        def start_in(c):
            b = c & 1
            return pltpu.async_copy(
                x_hbm.at[pl.ds(in_base + c * CH_IN_W, CH_IN_W)],
                in_bufs[b], sem_ins[b])

        def start_out(c):
            b = c & 1
            return pltpu.async_copy(
                out_bufs[b],
                out_hbm.at[pl.ds(out_base + c * CH_OUT_W, CH_OUT_W)],
                sem_outs[b])

        lanes4 = lax.iota(jnp.int32, 16) * 4

        def compute(c):
            b = c & 1
            iv = in_bufs[b]
            ov = out_bufs[b]

            def body(g, _):
                ot = g >> 6
                wi = g & 63
                base = (ot * 4096 + (wi & 1) * 64
                        + ((wi >> 1) & 3) * 1024 + (wi >> 3) * 128)
                i0 = lanes4 + base
                a = plsc.load_gather(iv, [i0])
                b_ = plsc.load_gather(iv, [i0 + 1])
                c_ = plsc.load_gather(iv, [i0 + 2])
                d_ = plsc.load_gather(iv, [i0 + 3])
                ov[pl.ds(g * 16, 16)] = (a + b_) + (c_ + d_)
                return 0

            lax.fori_loop(0, NGROUP, body, 0, unroll=4)

        in_h = [None] * NCHUNK
        out_h = [None] * NCHUNK
        in_h[0] = start_in(0)
        for c in range(NCHUNK):
            if c + 1 < NCHUNK:
                in_h[c + 1] = start_in(c + 1)
            in_h[c].wait()
            if c >= 2:
                out_h[c - 2].wait()
            compute(c)
            out_h[c] = start_out(c)
        out_h[NCHUNK - 2].wait()
        out_h[NCHUNK - 1].wait()

    return sc_kernel


_SC_KERNEL = _make_sc_kernel()


@jax.jit
def kernel(x):
    xf = (x.transpose(0, 2, 1)
          .reshape(B, 2, 8, F // 128, 128)
          .transpose(0, 1, 3, 2, 4)
          .reshape(IN_WORDS))
    of = _SC_KERNEL(xf)
    o5 = of.reshape(B, 2, OUT_F // 128, 8, 128).transpose(0, 1, 3, 2, 4)
    return o5.reshape(B, C, OUT_F).transpose(0, 2, 1)
